# Initial kernel scaffold; baseline (speedup 1.0000x reference)
import jax
import jax.numpy as jnp
from jax.experimental import pallas as pl


def kernel(inputs, word_emb_weight):
    raise NotImplementedError("write your pallas kernel here")


# speedup vs baseline: 1.4240x; 1.4240x over previous
import functools

import jax
import jax.numpy as jnp
from jax import lax
from jax.experimental import pallas as pl
from jax.experimental.pallas import tpu as pltpu
from jax.experimental.pallas import tpu_sc as plsc

_NC = 2
_NS = 16
_NW = _NC * _NS
_CHUNK = 128


def kernel(inputs, word_emb_weight):
    B, T = inputs.shape
    V, D = word_emb_weight.shape
    N = B * T
    per_w = N // _NW
    n_chunks = per_w // _CHUNK
    assert per_w * _NW == N and n_chunks * _CHUNK == per_w and n_chunks % 2 == 0

    idx3 = inputs.reshape(_NW, n_chunks, _CHUNK).astype(jnp.int32)

    mesh = plsc.VectorSubcoreMesh(
        core_axis_name="c", subcore_axis_name="s",
        num_cores=_NC, num_subcores=_NS)

    @functools.partial(
        pl.kernel,
        out_type=jax.ShapeDtypeStruct((N, D), jnp.float32),
        mesh=mesh,
        compiler_params=pltpu.CompilerParams(use_tc_tiling_on_sc=False),
        scratch_types=[
            pltpu.VMEM((n_chunks, _CHUNK), jnp.int32),
            pltpu.VMEM((2, _CHUNK, D), jnp.float32),
            pltpu.SemaphoreType.DMA,
            pltpu.SemaphoreType.DMA,
            pltpu.SemaphoreType.DMA,
            pltpu.SemaphoreType.DMA,
        ],
    )
    def emb(idx_hbm, table_hbm, out_hbm, idx_v, rows_v, g0, g1, s0, s1):
        wid = lax.axis_index("s") * _NC + lax.axis_index("c")
        base = wid * per_w
        pltpu.sync_copy(idx_hbm.at[wid], idx_v)
        gs = (g0, g1)
        ss = (s0, s1)

        def gather(g, b):
            return pltpu.make_async_copy(
                table_hbm.at[idx_v.at[g]], rows_v.at[b], gs[b])

        def store(g, b):
            return pltpu.make_async_copy(
                rows_v.at[b],
                out_hbm.at[pl.ds(base + g * _CHUNK, _CHUNK)],
                ss[b])

        gather(0, 0).start()
        gather(1, 1).start()

        def body(p, carry):
            for b in range(2):
                g = p * 2 + b
                gather(g, b).wait()
                store(g, b).start()

                @pl.when(g + 2 < n_chunks)
                def _():
                    store(g, b).wait()
                    gather(g + 2, b).start()
            return carry

        lax.fori_loop(0, n_chunks // 2, body, 0)
        for b in range(2):
            store(n_chunks - 2 + b, b).wait()

    out = emb(idx3, word_emb_weight)
    return out.reshape(B, T, D)
